# Initial kernel scaffold; baseline (speedup 1.0000x reference)
import jax
import jax.numpy as jnp
from jax.experimental import pallas as pl


def kernel(input):
    raise NotImplementedError("write your pallas kernel here")


# speedup vs baseline: 9.7590x; 9.7590x over previous
import functools

import jax
import jax.numpy as jnp
from jax import lax
from jax.experimental import pallas as pl
from jax.experimental.pallas import tpu as pltpu
from jax.experimental.pallas import tpu_sc as plsc

N_ROWS = 64
N_COLS = 32768
L = 16
CHUNKS = N_COLS // L
NUM_CORES = 2
NUM_SUBCORES = 16
NW = NUM_CORES * NUM_SUBCORES
ROWS_PER_W = N_ROWS // NW
N_ITERS = 26

_mesh = plsc.VectorSubcoreMesh(
    core_axis_name="c", subcore_axis_name="s",
    num_cores=NUM_CORES, num_subcores=NUM_SUBCORES,
)


@functools.partial(
    pl.kernel,
    out_type=jax.ShapeDtypeStruct((N_ROWS, N_COLS), jnp.float32),
    mesh=_mesh,
    scratch_types=[pltpu.VMEM((ROWS_PER_W, N_COLS), jnp.float32)],
    compiler_params=pltpu.CompilerParams(needs_layout_passes=False),
)
def _sparsemax_sc(x_hbm, out_hbm, buf):
    wid = lax.axis_index("c") * NUM_SUBCORES + lax.axis_index("s")
    base = wid * ROWS_PER_W
    pltpu.sync_copy(x_hbm.at[pl.ds(base, ROWS_PER_W)], buf)

    for r in range(ROWS_PER_W):
        def max_body(i, acc):
            return jnp.maximum(acc, buf[r, pl.ds(i * L, L)])

        acc = lax.fori_loop(1, CHUNKS, max_body, buf[r, pl.ds(0, L)],
                            unroll=8)
        m = jnp.broadcast_to(jnp.max(acc), (L,))

        def cs_pass(t):
            def body(i, carry):
                s_acc, c_acc = carry
                v = buf[r, pl.ds(i * L, L)]
                msk = v > t
                return (s_acc + jnp.where(msk, v, 0.0),
                        c_acc + jnp.where(msk, 1.0, 0.0))

            z = jnp.zeros((L,), jnp.float32)
            s_acc, c_acc = lax.fori_loop(0, CHUNKS, body, (z, z), unroll=8)
            return (jnp.broadcast_to(jnp.sum(s_acc), (L,)),
                    jnp.broadcast_to(jnp.sum(c_acc), (L,)))

        def bis_body(_, carry):
            lo, hi = carry
            t = 0.5 * (lo + hi)
            s, c = cs_pass(t)
            f = s - t * c - 1.0
            tnew = (s - 1.0) / jnp.maximum(c, 1.0)
            hi = jnp.where(f > 0.0, hi, t)
            lo = jnp.minimum(jnp.maximum(lo, tnew), hi)
            return lo, hi

        lo, hi = lax.fori_loop(0, N_ITERS, bis_body, (m - 1.0, m))

        s, c = cs_pass(lo)
        tau = jnp.where(c > 0.0, (s - 1.0) / jnp.maximum(c, 1.0), lo)

        def out_body(i, carry):
            v = buf[r, pl.ds(i * L, L)]
            buf[r, pl.ds(i * L, L)] = jnp.maximum(v - carry, 0.0)
            return carry

        lax.fori_loop(0, CHUNKS, out_body, tau, unroll=8)

    pltpu.sync_copy(buf, out_hbm.at[pl.ds(base, ROWS_PER_W)])


def kernel(input):
    return _sparsemax_sc(input)


# speedup vs baseline: 21.8963x; 2.2437x over previous
import functools

import jax
import jax.numpy as jnp
from jax import lax
from jax.experimental import pallas as pl
from jax.experimental.pallas import tpu as pltpu
from jax.experimental.pallas import tpu_sc as plsc

N_ROWS = 64
N_COLS = 32768
L = 16
CHUNKS = N_COLS // L
NUM_CORES = 2
NUM_SUBCORES = 16
NW = NUM_CORES * NUM_SUBCORES
ROWS_PER_W = N_ROWS // NW
N_ITERS = 26

_mesh = plsc.VectorSubcoreMesh(
    core_axis_name="c", subcore_axis_name="s",
    num_cores=NUM_CORES, num_subcores=NUM_SUBCORES,
)


@functools.partial(
    pl.kernel,
    out_type=jax.ShapeDtypeStruct((N_ROWS, N_COLS), jnp.float32),
    mesh=_mesh,
    scratch_types=[
        pltpu.VMEM((ROWS_PER_W, N_COLS), jnp.float32),
        pltpu.VMEM((N_COLS + L,), jnp.float32),
    ],
    compiler_params=pltpu.CompilerParams(needs_layout_passes=False),
)
def _sparsemax_sc(x_hbm, out_hbm, buf, cand):
    wid = lax.axis_index("c") * NUM_SUBCORES + lax.axis_index("s")
    base = wid * ROWS_PER_W
    pltpu.sync_copy(x_hbm.at[pl.ds(base, ROWS_PER_W)], buf)

    for r in range(ROWS_PER_W):
        def max_body(i, acc):
            return jnp.maximum(acc, buf[r, pl.ds(i * L, L)])

        acc = lax.fori_loop(1, CHUNKS, max_body, buf[r, pl.ds(0, L)],
                            unroll=8)
        m = jnp.broadcast_to(jnp.max(acc), (L,))
        lo0 = m - 1.0

        def compact_body(i, off):
            v = buf[r, pl.ds(i * L, L)]
            msk = v > lo0
            plsc.store_compressed(cand.at[pl.ds(off, L)], v, mask=msk)
            pc = plsc.all_reduce_population_count(msk)
            return off + pc[0]

        off = lax.fori_loop(0, CHUNKS, compact_body, jnp.int32(0),
                            unroll=4)
        plsc.store_compressed(cand.at[pl.ds(off, L)], lo0,
                              mask=jnp.ones((L,), jnp.bool_))
        nch = off // L + 1

        def cs_pass(t):
            def body(j, carry):
                s_acc, c_acc = carry
                v = cand[pl.ds(j * L, L)]
                msk = v > t
                return (s_acc + jnp.where(msk, v, 0.0),
                        c_acc + jnp.where(msk, 1.0, 0.0))

            z = jnp.zeros((L,), jnp.float32)
            s_acc, c_acc = lax.fori_loop(0, nch, body, (z, z))
            return (jnp.broadcast_to(jnp.sum(s_acc), (L,)),
                    jnp.broadcast_to(jnp.sum(c_acc), (L,)))

        def bis_body(_, carry):
            lo, hi = carry
            t = 0.5 * (lo + hi)
            s, c = cs_pass(t)
            f = s - t * c - 1.0
            tnew = (s - 1.0) / jnp.maximum(c, 1.0)
            hi = jnp.where(f > 0.0, hi, t)
            lo = jnp.minimum(jnp.maximum(lo, tnew), hi)
            return lo, hi

        lo, hi = lax.fori_loop(0, N_ITERS, bis_body, (lo0, m))

        s, c = cs_pass(lo)
        tau = jnp.where(c > 0.0, (s - 1.0) / jnp.maximum(c, 1.0), lo)

        def out_body(i, carry):
            v = buf[r, pl.ds(i * L, L)]
            buf[r, pl.ds(i * L, L)] = jnp.maximum(v - carry, 0.0)
            return carry

        lax.fori_loop(0, CHUNKS, out_body, tau, unroll=8)

    pltpu.sync_copy(buf, out_hbm.at[pl.ds(base, ROWS_PER_W)])


def kernel(input):
    return _sparsemax_sc(input)
